# Initial kernel scaffold; baseline (speedup 1.0000x reference)
import jax
import jax.numpy as jnp
from jax.experimental import pallas as pl


def kernel(x, table):
    raise NotImplementedError("write your pallas kernel here")


# speedup vs baseline: 1.6343x; 1.6343x over previous
import functools

import jax
import jax.numpy as jnp
from jax import lax
from jax.experimental import pallas as pl
from jax.experimental.pallas import tpu as pltpu
from jax.experimental.pallas import tpu_sc as plsc

_PAD_IDX = 1
_LANES = 16
_CHUNK = 16


def _pe_table(L, D):
    pos = jnp.arange(L, dtype=jnp.float32)[:, None]
    _2i = jnp.arange(0, D, 2, dtype=jnp.float32)
    angle = pos / jnp.power(10000.0, _2i / D)
    pe = jnp.zeros((L, D), dtype=jnp.float32)
    pe = pe.at[:, 0::2].set(jnp.sin(angle))
    pe = pe.at[:, 1::2].set(jnp.cos(angle))
    return pe


@functools.lru_cache(maxsize=None)
def _make_sc_embed(N, L, D):
    info = plsc.get_sparse_core_info()
    NC, NS = info.num_cores, info.num_subcores
    NW = NC * NS
    assert N % NW == 0
    bpw = N // NW
    assert bpw % _CHUNK == 0 and L % bpw == 0 and D % _LANES == 0
    nchunks = bpw // _CHUNK
    mesh = plsc.VectorSubcoreMesh(core_axis_name="c", subcore_axis_name="s")

    def body(x_hbm, pe_hbm, tbl_hbm, out_hbm, idx2, rows, pev, gsem):
        wid = lax.axis_index("s") * NC + lax.axis_index("c")
        base = wid * bpw
        pebase = base % L

        def chunk(c, carry):
            rb = base + c * _CHUNK
            pltpu.sync_copy(x_hbm.at[pl.ds(rb, _CHUNK)], idx2.at[c])
            pltpu.async_copy(tbl_hbm.at[idx2.at[c]], rows, gsem).wait()
            pltpu.sync_copy(pe_hbm.at[pl.ds(pebase + c * _CHUNK, _CHUNK)], pev)
            idxv = idx2[c, :]
            m = jnp.where(idxv == _PAD_IDX, 0.0, 1.0).astype(jnp.float32)
            for r in range(_CHUNK):
                mrow = jnp.full((_LANES,), m[r], jnp.float32)

                def jbody(j, c2, mrow=mrow, r=r):
                    sl = pl.ds(j * _LANES, _LANES)
                    rows[r, sl] = rows[r, sl] * mrow + pev[r, sl]
                    return c2

                lax.fori_loop(0, D // _LANES, jbody, 0)
            pltpu.sync_copy(rows, out_hbm.at[pl.ds(rb, _CHUNK)])
            return carry

        lax.fori_loop(0, nchunks, chunk, 0)

    return pl.kernel(
        body,
        mesh=mesh,
        out_type=jax.ShapeDtypeStruct((N, D), jnp.float32),
        scratch_types=[
            pltpu.VMEM((nchunks, _CHUNK), jnp.int32),
            pltpu.VMEM((_CHUNK, D), jnp.float32),
            pltpu.VMEM((_CHUNK, D), jnp.float32),
            pltpu.SemaphoreType.DMA,
        ],
    )


def kernel(x, table):
    B, L = x.shape
    _, D = table.shape
    pe = _pe_table(L, D)
    out = _make_sc_embed(B * L, L, D)(x.reshape(-1), pe, table)
    return out.reshape(B, L, D)


# speedup vs baseline: 2.5637x; 1.5687x over previous
import functools

import jax
import jax.numpy as jnp
from jax import lax
from jax.experimental import pallas as pl
from jax.experimental.pallas import tpu as pltpu
from jax.experimental.pallas import tpu_sc as plsc

_PAD_IDX = 1
_LANES = 16
_CHUNK = 16
_NBUF = 2


def _pe_table(L, D):
    pos = jnp.arange(L, dtype=jnp.float32)[:, None]
    _2i = jnp.arange(0, D, 2, dtype=jnp.float32)
    angle = pos / jnp.power(10000.0, _2i / D)
    pe = jnp.zeros((L, D), dtype=jnp.float32)
    pe = pe.at[:, 0::2].set(jnp.sin(angle))
    pe = pe.at[:, 1::2].set(jnp.cos(angle))
    return pe


@functools.lru_cache(maxsize=None)
def _make_sc_embed(N, L, D):
    info = plsc.get_sparse_core_info()
    NC, NS = info.num_cores, info.num_subcores
    NW = NC * NS
    assert N % NW == 0
    bpw = N // NW
    assert bpw % (_CHUNK * _NBUF) == 0 and L % bpw == 0 and D % _LANES == 0
    nchunks = bpw // _CHUNK
    mesh = plsc.VectorSubcoreMesh(core_axis_name="c", subcore_axis_name="s")

    def body(x3_hbm, pe_hbm, tbl_hbm, out_hbm,
             idx2, rows, pev, obuf, gs0, gs1, ps0, ps1, os0, os1):
        gsems = (gs0, gs1)
        psems = (ps0, ps1)
        osems = (os0, os1)
        wid = lax.axis_index("s") * NC + lax.axis_index("c")
        base = wid * bpw
        pebase = base % L

        pltpu.sync_copy(x3_hbm.at[wid], idx2)

        def issue_in(c, b):
            pltpu.async_copy(tbl_hbm.at[idx2.at[c]], rows.at[b], gsems[b])
            pltpu.async_copy(pe_hbm.at[pl.ds(pebase + c * _CHUNK, _CHUNK)],
                             pev.at[b], psems[b])

        issue_in(0, 0)
        issue_in(1, 1)

        def outer(c0, carry):
            for b in range(_NBUF):
                c = _NBUF * c0 + b
                pltpu.make_async_copy(
                    tbl_hbm.at[idx2.at[c]], rows.at[b], gsems[b]).wait()
                pltpu.make_async_copy(
                    pe_hbm.at[pl.ds(pebase + c * _CHUNK, _CHUNK)],
                    pev.at[b], psems[b]).wait()

                @pl.when(c0 > 0)
                def _wait_out():
                    pltpu.make_async_copy(
                        obuf.at[b],
                        out_hbm.at[pl.ds(base + (c - _NBUF) * _CHUNK, _CHUNK)],
                        osems[b]).wait()

                idxv = idx2[c, :]
                m = jnp.where(idxv == _PAD_IDX, 0.0, 1.0).astype(jnp.float32)
                for r in range(_CHUNK):
                    mrow = jnp.full((_LANES,), m[r], jnp.float32)

                    def jbody(j, c2, b=b, r=r, mrow=mrow):
                        sl = pl.ds(j * _LANES, _LANES)
                        obuf[b, r, sl] = rows[b, r, sl] * mrow + pev[b, r, sl]
                        return c2

                    lax.fori_loop(0, D // _LANES, jbody, 0, unroll=8)

                @pl.when(c + _NBUF < nchunks)
                def _prefetch():
                    issue_in(c + _NBUF, b)

                pltpu.async_copy(
                    obuf.at[b],
                    out_hbm.at[pl.ds(base + c * _CHUNK, _CHUNK)], osems[b])
            return carry

        lax.fori_loop(0, nchunks // _NBUF, outer, 0)

        for b in range(_NBUF):
            c = nchunks - _NBUF + b
            pltpu.make_async_copy(
                obuf.at[b],
                out_hbm.at[pl.ds(base + c * _CHUNK, _CHUNK)], osems[b]).wait()

    return pl.kernel(
        body,
        mesh=mesh,
        out_type=jax.ShapeDtypeStruct((N, D), jnp.float32),
        scratch_types=[
            pltpu.VMEM((N // NW // _CHUNK, _CHUNK), jnp.int32),
            pltpu.VMEM((_NBUF, _CHUNK, D), jnp.float32),
            pltpu.VMEM((_NBUF, _CHUNK, D), jnp.float32),
            pltpu.VMEM((_NBUF, _CHUNK, D), jnp.float32),
            pltpu.SemaphoreType.DMA,
            pltpu.SemaphoreType.DMA,
            pltpu.SemaphoreType.DMA,
            pltpu.SemaphoreType.DMA,
            pltpu.SemaphoreType.DMA,
            pltpu.SemaphoreType.DMA,
        ],
    )


def kernel(x, table):
    B, L = x.shape
    _, D = table.shape
    pe = _pe_table(L, D)
    info = plsc.get_sparse_core_info()
    NW = info.num_cores * info.num_subcores
    x3 = x.reshape(NW, (B * L) // (NW * _CHUNK), _CHUNK)
    out = _make_sc_embed(B * L, L, D)(x3, pe, table)
    return out.reshape(B, L, D)


# speedup vs baseline: 3.6692x; 1.4312x over previous
import functools

import jax
import jax.numpy as jnp
import numpy as np
from jax import lax
from jax.experimental import pallas as pl
from jax.experimental.pallas import tpu as pltpu
from jax.experimental.pallas import tpu_sc as plsc

_PAD_IDX = 1
_LANES = 16
_CHUNK = 16
_NBUF = 2


@functools.lru_cache(maxsize=None)
def _pe_table(L, D):
    pos = np.arange(L, dtype=np.float32)[:, None]
    _2i = np.arange(0, D, 2, dtype=np.float32)
    angle = (pos / np.power(10000.0, _2i / np.float32(D))).astype(np.float32)
    pe = np.stack([np.sin(angle), np.cos(angle)], axis=-1).reshape(L, D)
    return jnp.asarray(pe.astype(np.float32))


@functools.lru_cache(maxsize=None)
def _make_sc_embed(N, L, D):
    info = plsc.get_sparse_core_info()
    NC, NS = info.num_cores, info.num_subcores
    NW = NC * NS
    assert N % NW == 0
    bpw = N // NW
    assert bpw % (_CHUNK * _NBUF) == 0 and L % bpw == 0 and D % _LANES == 0
    nchunks = bpw // _CHUNK
    mesh = plsc.VectorSubcoreMesh(core_axis_name="c", subcore_axis_name="s")

    def body(x3_hbm, pe_hbm, tbl_hbm, out_hbm,
             idx2, rows, pev, obuf, gs0, gs1, ps0, ps1, os0, os1):
        gsems = (gs0, gs1)
        psems = (ps0, ps1)
        osems = (os0, os1)
        wid = lax.axis_index("s") * NC + lax.axis_index("c")
        base = wid * bpw
        pebase = base % L

        pltpu.sync_copy(x3_hbm.at[wid], idx2)

        def issue_in(c, b):
            pltpu.async_copy(tbl_hbm.at[idx2.at[c]], rows.at[b], gsems[b])
            pltpu.async_copy(pe_hbm.at[pl.ds(pebase + c * _CHUNK, _CHUNK)],
                             pev.at[b], psems[b])

        issue_in(0, 0)
        issue_in(1, 1)

        def outer(c0, carry):
            for b in range(_NBUF):
                c = _NBUF * c0 + b
                pltpu.make_async_copy(
                    tbl_hbm.at[idx2.at[c]], rows.at[b], gsems[b]).wait()
                pltpu.make_async_copy(
                    pe_hbm.at[pl.ds(pebase + c * _CHUNK, _CHUNK)],
                    pev.at[b], psems[b]).wait()

                @pl.when(c0 > 0)
                def _wait_out():
                    pltpu.make_async_copy(
                        obuf.at[b],
                        out_hbm.at[pl.ds(base + (c - _NBUF) * _CHUNK, _CHUNK)],
                        osems[b]).wait()

                idxv = idx2[c, :]
                m = jnp.where(idxv == _PAD_IDX, 0.0, 1.0).astype(jnp.float32)
                for r in range(_CHUNK):
                    mrow = jnp.full((_LANES,), m[r], jnp.float32)

                    def jbody(j, c2, b=b, r=r, mrow=mrow):
                        sl = pl.ds(j * _LANES, _LANES)
                        obuf[b, r, sl] = rows[b, r, sl] * mrow + pev[b, r, sl]
                        return c2

                    lax.fori_loop(0, D // _LANES, jbody, 0, unroll=8)

                @pl.when(c + _NBUF < nchunks)
                def _prefetch():
                    issue_in(c + _NBUF, b)

                pltpu.async_copy(
                    obuf.at[b],
                    out_hbm.at[pl.ds(base + c * _CHUNK, _CHUNK)], osems[b])
            return carry

        lax.fori_loop(0, nchunks // _NBUF, outer, 0)

        for b in range(_NBUF):
            c = nchunks - _NBUF + b
            pltpu.make_async_copy(
                obuf.at[b],
                out_hbm.at[pl.ds(base + c * _CHUNK, _CHUNK)], osems[b]).wait()

    return pl.kernel(
        body,
        mesh=mesh,
        out_type=jax.ShapeDtypeStruct((N, D), jnp.float32),
        scratch_types=[
            pltpu.VMEM((N // NW // _CHUNK, _CHUNK), jnp.int32),
            pltpu.VMEM((_NBUF, _CHUNK, D), jnp.float32),
            pltpu.VMEM((_NBUF, _CHUNK, D), jnp.float32),
            pltpu.VMEM((_NBUF, _CHUNK, D), jnp.float32),
            pltpu.SemaphoreType.DMA,
            pltpu.SemaphoreType.DMA,
            pltpu.SemaphoreType.DMA,
            pltpu.SemaphoreType.DMA,
            pltpu.SemaphoreType.DMA,
            pltpu.SemaphoreType.DMA,
        ],
    )


def kernel(x, table):
    B, L = x.shape
    _, D = table.shape
    pe = _pe_table(L, D)
    info = plsc.get_sparse_core_info()
    NW = info.num_cores * info.num_subcores
    x3 = x.reshape(NW, (B * L) // (NW * _CHUNK), _CHUNK)
    out = _make_sc_embed(B * L, L, D)(x3, pe, table)
    return out.reshape(B, L, D)
